# baseline (speedup 1.0000x reference)
import functools

import jax
import jax.numpy as jnp
from jax import lax
from jax.experimental import pallas as pl
from jax.experimental.pallas import tpu as pltpu
from jax.experimental.pallas import tpu_sc as plsc

VOCAB = 1000000
DIM = 64
BATCH = 16384
FIELDS = 100

NC = 2
NS = 16
NW = NC * NS

B = BATCH * FIELDS
B_PER_W = B // NW
IDXW = 128
CHUNK = IDXW
NBUF = 4
NCHUNK = B_PER_W // CHUNK
IROWS = B_PER_W // IDXW


def _gather_body(table_hbm, idx_hbm, out_hbm, idx_all, *scratch):
    rows = scratch[:NBUF]
    gsems = scratch[NBUF:2 * NBUF]
    spmem, ssem, dsem = scratch[2 * NBUF:]

    cid = lax.axis_index("c")
    sid = lax.axis_index("s")
    wid = sid * NC + cid
    base = wid * B_PER_W
    sp = spmem.at[pl.ds(pl.multiple_of(sid * CHUNK, 8), CHUNK)]

    pltpu.sync_copy(
        idx_hbm.at[pl.ds(pl.multiple_of(wid * IROWS, 8), IROWS)], idx_all
    )

    def fire_g(g, b):
        pltpu.async_copy(table_hbm.at[idx_all.at[g]], rows[b], gsems[b])

    def wait_g(b):
        pltpu.make_async_copy(
            out_hbm.at[pl.ds(0, CHUNK)], rows[b], gsems[b]
        ).wait()

    def spill(b):
        pltpu.async_copy(rows[b], sp, ssem)
        pltpu.make_async_copy(rows[b], sp, ssem).wait()

    def fire_d(g):
        pltpu.async_copy(sp, out_hbm.at[pl.ds(base + g * CHUNK, CHUNK)], dsem)

    def wait_d():
        pltpu.make_async_copy(sp, out_hbm.at[pl.ds(0, CHUNK)], dsem).wait()

    for b in range(NBUF):
        fire_g(b, b)
    wait_g(0)
    spill(0)
    fire_d(0)
    fire_g(NBUF, 0)
    for b in range(1, NBUF):
        wait_g(b)
        wait_d()
        spill(b)
        fire_d(b)
        fire_g(b + NBUF, b)

    def body(j, carry):
        g0 = NBUF * j
        for b in range(NBUF):
            g = g0 + b
            wait_g(b)
            wait_d()
            spill(b)
            fire_d(g)
            fire_g(g + NBUF, b)
        return carry

    lax.fori_loop(1, NCHUNK // NBUF - 1, body, 0)

    g0 = NCHUNK - NBUF
    for b in range(NBUF):
        wait_g(b)
        wait_d()
        spill(b)
        fire_d(g0 + b)
    wait_d()


@functools.partial(jax.jit, static_argnames=())
def kernel(input_ids, weight):
    flat = input_ids.reshape(B // IDXW, IDXW).astype(jnp.int32)
    mesh = plsc.VectorSubcoreMesh(core_axis_name="c", subcore_axis_name="s")
    scratch = (
        [pltpu.VMEM((CHUNK, DIM), jnp.float32)] * NBUF
        + [pltpu.SemaphoreType.DMA] * NBUF
        + [
            pltpu.VMEM_SHARED((NS * CHUNK, DIM), jnp.float32),
            pltpu.SemaphoreType.DMA,
            pltpu.SemaphoreType.DMA,
        ]
    )
    out = pl.kernel(
        _gather_body,
        out_type=jax.ShapeDtypeStruct((B, DIM), jnp.float32),
        mesh=mesh,
        scratch_types=[pltpu.VMEM((IROWS, IDXW), jnp.int32)] + scratch,
        compiler_params=pltpu.CompilerParams(use_tc_tiling_on_sc=False),
    )(weight, flat)
    return out.reshape(BATCH, FIELDS, DIM)


# speedup vs baseline: 1.0636x; 1.0636x over previous
import functools

import jax
import jax.numpy as jnp
from jax import lax
from jax.experimental import pallas as pl
from jax.experimental.pallas import tpu as pltpu
from jax.experimental.pallas import tpu_sc as plsc

VOCAB = 1000000
DIM = 64
BATCH = 16384
FIELDS = 100

NC = 2
NS = 16
NW = NC * NS

B = BATCH * FIELDS
B_PER_W = B // NW
IDXW = 128
KGATH = 2
NBUF = 4
CHUNK = IDXW * KGATH
NCHUNK = B_PER_W // CHUNK
IROWS = B_PER_W // IDXW
IHALF = IROWS // 2
GHALF = NCHUNK // 2


def _gather_body(table_hbm, idx_hbm, out_hbm, idx_all, *scratch):
    rows = scratch[:NBUF]
    gsems = scratch[NBUF:2 * NBUF]
    wsems = scratch[2 * NBUF:3 * NBUF]
    isem0 = scratch[3 * NBUF]
    isem1 = scratch[3 * NBUF + 1]

    wid = lax.axis_index("s") * NC + lax.axis_index("c")
    base = wid * B_PER_W
    ibase = pl.multiple_of(wid * IROWS, 8)

    pltpu.async_copy(
        idx_hbm.at[pl.ds(ibase, IHALF)], idx_all.at[pl.ds(0, IHALF)], isem0
    )
    pltpu.async_copy(
        idx_hbm.at[pl.ds(ibase + IHALF, IHALF)],
        idx_all.at[pl.ds(IHALF, IHALF)],
        isem1,
    )
    pltpu.make_async_copy(
        idx_hbm.at[pl.ds(0, IHALF)], idx_all.at[pl.ds(0, IHALF)], isem0
    ).wait()

    def fire_gathers(g, b):
        for k in range(KGATH):
            pltpu.async_copy(
                table_hbm.at[idx_all.at[g * KGATH + k]],
                rows[b].at[pl.ds(k * IDXW, IDXW)],
                gsems[b],
            )

    def drain_gathers(b):
        pltpu.make_async_copy(
            out_hbm.at[pl.ds(0, CHUNK)], rows[b], gsems[b]
        ).wait()

    def fire_wb(g, b):
        pltpu.async_copy(
            rows[b], out_hbm.at[pl.ds(base + g * CHUNK, CHUNK)], wsems[b]
        )

    def drain_wb(b):
        pltpu.make_async_copy(
            rows[b], out_hbm.at[pl.ds(0, CHUNK)], wsems[b]
        ).wait()

    for b in range(NBUF):
        fire_gathers(b, b)

    def make_body(extra_wait):
        def body(j, carry):
            g0 = NBUF * j
            if extra_wait:
                pltpu.make_async_copy(
                    idx_hbm.at[pl.ds(0, IHALF)],
                    idx_all.at[pl.ds(0, IHALF)],
                    isem1,
                ).wait()
            for b in range(NBUF):
                drain_gathers(b)
                fire_wb(g0 + b, b)
            for b in range(NBUF):
                drain_wb(b)
                fire_gathers(g0 + b + NBUF, b)
            return carry
        return body

    half_groups = GHALF // NBUF
    lax.fori_loop(0, half_groups - 1, make_body(False), 0)
    make_body(True)(half_groups - 1, 0)
    lax.fori_loop(half_groups, NCHUNK // NBUF - 1, make_body(False), 0)

    g0 = NCHUNK - NBUF
    for b in range(NBUF):
        drain_gathers(b)
        fire_wb(g0 + b, b)
    for b in range(NBUF):
        drain_wb(b)


@functools.partial(jax.jit, static_argnames=())
def kernel(input_ids, weight):
    flat = input_ids.reshape(B // IDXW, IDXW).astype(jnp.int32)
    mesh = plsc.VectorSubcoreMesh(core_axis_name="c", subcore_axis_name="s")
    scratch = (
        [pltpu.VMEM((CHUNK, DIM), jnp.float32)] * NBUF
        + [pltpu.SemaphoreType.DMA] * (2 * NBUF + 2)
    )
    out = pl.kernel(
        _gather_body,
        out_type=jax.ShapeDtypeStruct((B, DIM), jnp.float32),
        mesh=mesh,
        scratch_types=[pltpu.VMEM((IROWS, IDXW), jnp.int32)] + scratch,
        compiler_params=pltpu.CompilerParams(use_tc_tiling_on_sc=False),
    )(weight, flat)
    return out.reshape(BATCH, FIELDS, DIM)
